# baseline (speedup 1.0000x reference)
import functools
import math

import jax
import jax.numpy as jnp
import numpy as np
from jax import lax
from jax.experimental import pallas as pl
from jax.experimental.pallas import tpu as pltpu
from jax.experimental.pallas import tpu_sc as plsc

_NUM_BUCKETS = 32
_NUM_HEADS = 16
_LQ = 2048
_LK = 2048
_MAX_DIST = 128
_SHIFTS = 8
_TBL = 4096
_HALF = _LQ // 2
_FLIGHT = 16


def _bucket_table() -> np.ndarray:
    d = np.arange(-(_LQ - 1), _LK, dtype=np.int32)
    rel_buckets = (d > 0).astype(np.int32) * (_NUM_BUCKETS // 2)
    rp = np.abs(d)
    max_exact = _NUM_BUCKETS // 4
    safe_rp = np.maximum(rp.astype(np.float32), np.float32(1e-9))
    large = max_exact + (
        np.log(safe_rp / max_exact)
        / math.log(_MAX_DIST / max_exact)
        * (_NUM_BUCKETS // 2 - max_exact)
    ).astype(np.int32)
    large = np.minimum(large, _NUM_BUCKETS // 2 - 1)
    return rel_buckets + np.where(rp < max_exact, rp, large)


def _shifted_index_table() -> np.ndarray:
    bucket = _bucket_table()
    u = np.arange(_TBL)
    out = np.empty((_SHIFTS, _TBL), dtype=np.int32)
    for k in range(_SHIFTS):
        dd = np.minimum(u + k, _LQ + _LK - 2)
        out[_SHIFTS - 1 - k] = bucket[dd]
    return out.reshape(-1)


_IDX_CONST = _shifted_index_table()


@functools.lru_cache(maxsize=1)
def _build_fill_kernel():
    mesh = plsc.VectorSubcoreMesh(core_axis_name="c", subcore_axis_name="s")
    return functools.partial(
        pl.kernel,
        out_type=jax.ShapeDtypeStruct((_NUM_HEADS * _LQ, _LK), jnp.float32),
        mesh=mesh,
        scratch_types=[
            pltpu.VMEM((_NUM_HEADS * _NUM_BUCKETS,), jnp.float32),
            pltpu.VMEM((_SHIFTS * _TBL,), jnp.int32),
            pltpu.VMEM((_SHIFTS * _TBL,), jnp.float32),
            pltpu.VMEM((8, _LK), jnp.float32),
            pltpu.VMEM((8, _LK), jnp.float32),
            pltpu.SemaphoreType.DMA,
        ],
        compiler_params=pltpu.CompilerParams(needs_layout_passes=False),
    )(_t5_bias_fill)


def _t5_bias_fill(wt_hbm, idx_hbm, out_hbm, w_v, idx_v, table_v, st_a, st_b, sem):
    head = lax.axis_index("s")
    half = lax.axis_index("c")
    pltpu.sync_copy(wt_hbm, w_v)
    pltpu.sync_copy(idx_hbm, idx_v)
    hbase = head * _NUM_BUCKETS

    with jax.named_scope("tbl_build"):

        @pl.loop(0, _SHIFTS * _TBL // 16)
        def _build(t):
            base = t * 16
            iv = idx_v[pl.ds(base, 16)]
            table_v[pl.ds(base, 16)] = plsc.load_gather(w_v, [iv + hbase])

    i0 = half * _HALF
    row0 = head * _LQ + i0

    def _fill_stage(stg, w0):
        @pl.loop(0, _LK // 16)
        def _cp(c):
            c16 = c * 16
            src = pl.multiple_of(w0 + c16, 8)
            for r in range(8):
                stg[r, pl.ds(c16, 16)] = table_v[pl.ds(r * _TBL + src, 16)]

    with jax.named_scope("row_fill"):

        @pl.loop(0, _HALF // 16)
        def _fill(g):
            s0 = g * 2
            w0a = (_LQ - 8) - i0 - s0 * 8
            _fill_stage(st_a, w0a)
            da = pltpu.async_copy(st_a, out_hbm.at[pl.ds(row0 + s0 * 8, 8), :], sem)
            _fill_stage(st_b, w0a - 8)
            db = pltpu.async_copy(st_b, out_hbm.at[pl.ds(row0 + s0 * 8 + 8, 8), :], sem)
            da.wait()
            db.wait()


def kernel(lq, lk, W):
    del lq, lk
    wt = W.astype(jnp.float32).T.reshape(-1)
    idx = jnp.asarray(_IDX_CONST)
    out = _build_fill_kernel()(wt, idx)
    return out.reshape(1, _NUM_HEADS, _LQ, _LK)


# speedup vs baseline: 2.7793x; 2.7793x over previous
import functools
import math

import jax
import jax.numpy as jnp
import numpy as np
from jax import lax
from jax.experimental import pallas as pl
from jax.experimental.pallas import tpu as pltpu
from jax.experimental.pallas import tpu_sc as plsc

_NUM_BUCKETS = 32
_NUM_HEADS = 16
_LQ = 2048
_LK = 2048
_MAX_DIST = 128
_SHIFTS = 8
_TBL = 4096
_HALF = _LQ // 2
_FLIGHT = 16


def _bucket_table() -> np.ndarray:
    d = np.arange(-(_LQ - 1), _LK, dtype=np.int32)
    rel_buckets = (d > 0).astype(np.int32) * (_NUM_BUCKETS // 2)
    rp = np.abs(d)
    max_exact = _NUM_BUCKETS // 4
    safe_rp = np.maximum(rp.astype(np.float32), np.float32(1e-9))
    large = max_exact + (
        np.log(safe_rp / max_exact)
        / math.log(_MAX_DIST / max_exact)
        * (_NUM_BUCKETS // 2 - max_exact)
    ).astype(np.int32)
    large = np.minimum(large, _NUM_BUCKETS // 2 - 1)
    return rel_buckets + np.where(rp < max_exact, rp, large)


def _shifted_index_table() -> np.ndarray:
    bucket = _bucket_table()
    u = np.arange(_TBL)
    out = np.empty((_SHIFTS, _TBL), dtype=np.int32)
    for k in range(_SHIFTS):
        dd = np.minimum(u + k, _LQ + _LK - 2)
        out[_SHIFTS - 1 - k] = bucket[dd]
    return out.reshape(-1)


_IDX_CONST = _shifted_index_table()


@functools.lru_cache(maxsize=1)
def _build_fill_kernel():
    mesh = plsc.VectorSubcoreMesh(core_axis_name="c", subcore_axis_name="s")
    return functools.partial(
        pl.kernel,
        out_type=jax.ShapeDtypeStruct((_NUM_HEADS * _LQ, _LK), jnp.float32),
        mesh=mesh,
        scratch_types=[
            pltpu.VMEM((_NUM_HEADS * _NUM_BUCKETS,), jnp.float32),
            pltpu.VMEM((_SHIFTS * _TBL,), jnp.int32),
            pltpu.VMEM((_SHIFTS * _TBL,), jnp.float32),
            pltpu.VMEM((8, _LK), jnp.float32),
            pltpu.VMEM((8, _LK), jnp.float32),
            pltpu.SemaphoreType.DMA,
        ],
        compiler_params=pltpu.CompilerParams(needs_layout_passes=False),
    )(_t5_bias_fill)


def _t5_bias_fill(wt_hbm, idx_hbm, out_hbm, w_v, idx_v, table_v, st_a, st_b, sem):
    head = lax.axis_index("s")
    half = lax.axis_index("c")
    pltpu.sync_copy(wt_hbm, w_v)
    pltpu.sync_copy(idx_hbm, idx_v)
    hbase = head * _NUM_BUCKETS

    with jax.named_scope("tbl_build"):

        @pl.loop(0, _SHIFTS * _TBL // 16)
        def _build(t):
            base = t * 16
            iv = idx_v[pl.ds(base, 16)]
            table_v[pl.ds(base, 16)] = plsc.load_gather(w_v, [iv + hbase])

    i0 = half * _HALF
    row0 = head * _LQ + i0

    def _fill_stage(stg, w0):
        @pl.loop(0, _LK // 16)
        def _cp(c):
            c16 = c * 16
            src = pl.multiple_of(w0 + c16, 8)
            vals = [table_v[pl.ds(r * _TBL + src, 16)] for r in range(8)]
            for r in range(8):
                stg[r, pl.ds(c16, 16)] = vals[r]

    with jax.named_scope("row_fill"):

        @pl.loop(0, _HALF // 16)
        def _fill(g):
            s0 = g * 2
            w0a = (_LQ - 8) - i0 - s0 * 8
            _fill_stage(st_a, w0a)
            da = pltpu.async_copy(st_a, out_hbm.at[pl.ds(row0 + s0 * 8, 8), :], sem)
            _fill_stage(st_b, w0a - 8)
            db = pltpu.async_copy(st_b, out_hbm.at[pl.ds(row0 + s0 * 8 + 8, 8), :], sem)
            da.wait()
            db.wait()


def kernel(lq, lk, W):
    del lq, lk
    wt = W.astype(jnp.float32).T.reshape(-1)
    idx = jnp.asarray(_IDX_CONST)
    out = _build_fill_kernel()(wt, idx)
    return out.reshape(1, _NUM_HEADS, _LQ, _LK)


# speedup vs baseline: 3.1833x; 1.1454x over previous
import functools
import math

import jax
import jax.numpy as jnp
import numpy as np
from jax import lax
from jax.experimental import pallas as pl
from jax.experimental.pallas import tpu as pltpu
from jax.experimental.pallas import tpu_sc as plsc

_NUM_BUCKETS = 32
_NUM_HEADS = 16
_LQ = 2048
_LK = 2048
_MAX_DIST = 128
_SHIFTS = 8
_TBL = 4096
_HALF = _LQ // 2
_FLIGHT = 16


def _bucket_table() -> np.ndarray:
    d = np.arange(-(_LQ - 1), _LK, dtype=np.int32)
    rel_buckets = (d > 0).astype(np.int32) * (_NUM_BUCKETS // 2)
    rp = np.abs(d)
    max_exact = _NUM_BUCKETS // 4
    safe_rp = np.maximum(rp.astype(np.float32), np.float32(1e-9))
    large = max_exact + (
        np.log(safe_rp / max_exact)
        / math.log(_MAX_DIST / max_exact)
        * (_NUM_BUCKETS // 2 - max_exact)
    ).astype(np.int32)
    large = np.minimum(large, _NUM_BUCKETS // 2 - 1)
    return rel_buckets + np.where(rp < max_exact, rp, large)


def _shifted_index_table() -> np.ndarray:
    bucket = _bucket_table()
    u = np.arange(_TBL)
    out = np.empty((_SHIFTS, _TBL), dtype=np.int32)
    for k in range(_SHIFTS):
        dd = np.minimum(u + k, _LQ + _LK - 2)
        out[_SHIFTS - 1 - k] = bucket[dd]
    return out.reshape(-1)


_IDX_CONST = _shifted_index_table()


@functools.lru_cache(maxsize=1)
def _build_fill_kernel():
    mesh = plsc.VectorSubcoreMesh(core_axis_name="c", subcore_axis_name="s")
    return functools.partial(
        pl.kernel,
        out_type=jax.ShapeDtypeStruct((_NUM_HEADS * _LQ, _LK), jnp.float32),
        mesh=mesh,
        scratch_types=[
            pltpu.VMEM((_NUM_HEADS * _NUM_BUCKETS,), jnp.float32),
            pltpu.VMEM((_SHIFTS * _TBL,), jnp.float32),
            pltpu.VMEM((8, _LK), jnp.float32),
            pltpu.VMEM((8, _LK), jnp.float32),
            pltpu.VMEM((8, _LK), jnp.float32),
            pltpu.VMEM((8, _LK), jnp.float32),
            pltpu.SemaphoreType.DMA,
        ],
        compiler_params=pltpu.CompilerParams(needs_layout_passes=False),
    )(_t5_bias_fill)


def _t5_bias_fill(wt_hbm, idx_hbm, out_hbm, w_v, table_v, s0, s1, s2, s3, sem):
    head = lax.axis_index("s")
    half = lax.axis_index("c")
    pltpu.sync_copy(wt_hbm, w_v)
    pltpu.sync_copy(idx_hbm, table_v)
    hbase = head * _NUM_BUCKETS

    with jax.named_scope("tbl_build"):

        @pl.loop(0, _SHIFTS * _TBL // 16)
        def _build(t):
            base = t * 16
            iv = plsc.bitcast(table_v[pl.ds(base, 16)], jnp.int32)
            table_v[pl.ds(base, 16)] = plsc.load_gather(w_v, [iv + hbase])

    i0 = half * _HALF
    row0 = head * _LQ + i0
    ring = (s0, s1, s2, s3)

    def _fill_stage(stg, w0):
        @pl.loop(0, _LK // 16)
        def _cp(c):
            c16 = c * 16
            src = pl.multiple_of(w0 + c16, 8)
            vals = [table_v[pl.ds(r * _TBL + src, 16)] for r in range(8)]
            for r in range(8):
                stg[r, pl.ds(c16, 16)] = vals[r]

    def _stripe(stg, s_loc):
        _fill_stage(stg, (_LQ - 8) - i0 - s_loc * 8)
        pltpu.async_copy(stg, out_hbm.at[pl.ds(row0 + s_loc * 8, 8), :], sem)

    with jax.named_scope("row_fill"):
        for b in range(4):
            _stripe(ring[b], b)

        @pl.loop(1, _HALF // 8 // 4)
        def _fill(g):
            for b in range(4):
                pltpu.make_async_copy(out_hbm.at[pl.ds(0, 8), :], ring[b], sem).wait()
                _stripe(ring[b], g * 4 + b)

        for b in range(4):
            pltpu.make_async_copy(out_hbm.at[pl.ds(0, 8), :], ring[b], sem).wait()


def kernel(lq, lk, W):
    del lq, lk
    wt = W.astype(jnp.float32).T.reshape(-1)
    idx = lax.bitcast_convert_type(jnp.asarray(_IDX_CONST), jnp.float32)
    out = _build_fill_kernel()(wt, idx)
    return out.reshape(1, _NUM_HEADS, _LQ, _LK)


# speedup vs baseline: 4.5106x; 1.4170x over previous
import functools
import math

import jax
import jax.numpy as jnp
import numpy as np
from jax import lax
from jax.experimental import pallas as pl
from jax.experimental.pallas import tpu as pltpu
from jax.experimental.pallas import tpu_sc as plsc

_NUM_BUCKETS = 32
_NUM_HEADS = 16
_LQ = 2048
_LK = 2048
_MAX_DIST = 128
_SHIFTS = 8
_TBL = 4096
_HALF = _LQ // 2
_FLIGHT = 16


def _bucket_table() -> np.ndarray:
    d = np.arange(-(_LQ - 1), _LK, dtype=np.int32)
    rel_buckets = (d > 0).astype(np.int32) * (_NUM_BUCKETS // 2)
    rp = np.abs(d)
    max_exact = _NUM_BUCKETS // 4
    safe_rp = np.maximum(rp.astype(np.float32), np.float32(1e-9))
    large = max_exact + (
        np.log(safe_rp / max_exact)
        / math.log(_MAX_DIST / max_exact)
        * (_NUM_BUCKETS // 2 - max_exact)
    ).astype(np.int32)
    large = np.minimum(large, _NUM_BUCKETS // 2 - 1)
    return rel_buckets + np.where(rp < max_exact, rp, large)


def _shifted_index_table() -> np.ndarray:
    bucket = _bucket_table()
    u = np.arange(_TBL)
    out = np.empty((_SHIFTS, _TBL), dtype=np.int32)
    for k in range(_SHIFTS):
        dd = np.minimum(u + k, _LQ + _LK - 2)
        out[_SHIFTS - 1 - k] = bucket[dd]
    return out.reshape(-1)


_IDX_CONST = _shifted_index_table()


@functools.lru_cache(maxsize=1)
def _build_fill_kernel():
    mesh = plsc.VectorSubcoreMesh(core_axis_name="c", subcore_axis_name="s")
    return functools.partial(
        pl.kernel,
        out_type=jax.ShapeDtypeStruct((_NUM_HEADS * _LQ, _LK), jnp.float32),
        mesh=mesh,
        scratch_types=[
            pltpu.VMEM((_NUM_HEADS * _NUM_BUCKETS,), jnp.float32),
            pltpu.VMEM((_SHIFTS * _TBL,), jnp.float32),
            pltpu.VMEM((8, _LK), jnp.float32),
            pltpu.VMEM((8, _LK), jnp.float32),
            pltpu.VMEM((8, _LK), jnp.float32),
            pltpu.VMEM((8, _LK), jnp.float32),
            pltpu.SemaphoreType.DMA,
        ],
        compiler_params=pltpu.CompilerParams(needs_layout_passes=False),
    )(_t5_bias_fill)


def _t5_bias_fill(wt_hbm, idx_hbm, out_hbm, w_v, table_v, s0, s1, s2, s3, sem):
    head = lax.axis_index("s")
    half = lax.axis_index("c")
    pltpu.sync_copy(wt_hbm, w_v)
    pltpu.sync_copy(idx_hbm, table_v)
    hbase = head * _NUM_BUCKETS

    with jax.named_scope("tbl_build"):

        @pl.loop(0, _SHIFTS * _TBL // 16)
        def _build(t):
            base = t * 16
            iv = plsc.bitcast(table_v[pl.ds(base, 16)], jnp.int32)
            table_v[pl.ds(base, 16)] = plsc.load_gather(w_v, [iv + hbase])

    i0 = half * _HALF
    row0 = head * _LQ + i0
    ring = (s0, s1, s2, s3)

    def _fill_stage(stg, w0):
        @plsc.parallel_loop(0, _LK // 16)
        def _cp(c):
            c16 = c * 16
            src = pl.multiple_of(w0 + c16, 8)
            vals = [table_v[pl.ds(r * _TBL + src, 16)] for r in range(8)]
            for r in range(8):
                stg[r, pl.ds(c16, 16)] = vals[r]

    def _stripe(stg, s_loc):
        _fill_stage(stg, (_LQ - 8) - i0 - s_loc * 8)
        pltpu.async_copy(stg, out_hbm.at[pl.ds(row0 + s_loc * 8, 8), :], sem)

    with jax.named_scope("row_fill"):
        for b in range(4):
            _stripe(ring[b], b)

        @pl.loop(1, _HALF // 8 // 4)
        def _fill(g):
            for b in range(4):
                pltpu.make_async_copy(out_hbm.at[pl.ds(0, 8), :], ring[b], sem).wait()
                _stripe(ring[b], g * 4 + b)

        for b in range(4):
            pltpu.make_async_copy(out_hbm.at[pl.ds(0, 8), :], ring[b], sem).wait()


def kernel(lq, lk, W):
    del lq, lk
    wt = W.astype(jnp.float32).T.reshape(-1)
    idx = lax.bitcast_convert_type(jnp.asarray(_IDX_CONST), jnp.float32)
    out = _build_fill_kernel()(wt, idx)
    return out.reshape(1, _NUM_HEADS, _LQ, _LK)


# speedup vs baseline: 4.9944x; 1.1073x over previous
import functools
import math

import jax
import jax.numpy as jnp
import numpy as np
from jax import lax
from jax.experimental import pallas as pl
from jax.experimental.pallas import tpu as pltpu
from jax.experimental.pallas import tpu_sc as plsc

_NUM_BUCKETS = 32
_NUM_HEADS = 16
_LQ = 2048
_LK = 2048
_MAX_DIST = 128
_SHIFTS = 8
_TBL = 4096
_HALF = _LQ // 2
_FLIGHT = 16


def _bucket_table() -> np.ndarray:
    d = np.arange(-(_LQ - 1), _LK, dtype=np.int32)
    rel_buckets = (d > 0).astype(np.int32) * (_NUM_BUCKETS // 2)
    rp = np.abs(d)
    max_exact = _NUM_BUCKETS // 4
    safe_rp = np.maximum(rp.astype(np.float32), np.float32(1e-9))
    large = max_exact + (
        np.log(safe_rp / max_exact)
        / math.log(_MAX_DIST / max_exact)
        * (_NUM_BUCKETS // 2 - max_exact)
    ).astype(np.int32)
    large = np.minimum(large, _NUM_BUCKETS // 2 - 1)
    return rel_buckets + np.where(rp < max_exact, rp, large)


def _shifted_index_table() -> np.ndarray:
    bucket = _bucket_table()
    u = np.arange(_TBL)
    out = np.empty((_SHIFTS, _TBL), dtype=np.int32)
    for k in range(_SHIFTS):
        dd = np.minimum(u + k, _LQ + _LK - 2)
        out[_SHIFTS - 1 - k] = bucket[dd]
    return out.reshape(-1)


_IDX_CONST = _shifted_index_table()


@functools.lru_cache(maxsize=1)
def _build_fill_kernel():
    mesh = plsc.VectorSubcoreMesh(core_axis_name="c", subcore_axis_name="s")
    return functools.partial(
        pl.kernel,
        out_type=jax.ShapeDtypeStruct((_NUM_HEADS * _LQ, _LK), jnp.float32),
        mesh=mesh,
        scratch_types=[
            pltpu.VMEM((_NUM_HEADS * _NUM_BUCKETS,), jnp.float32),
            pltpu.VMEM((_SHIFTS * _TBL,), jnp.float32),
            pltpu.VMEM((8, _LK), jnp.float32),
            pltpu.VMEM((8, _LK), jnp.float32),
            pltpu.VMEM((8, _LK), jnp.float32),
            pltpu.VMEM((8, _LK), jnp.float32),
            pltpu.SemaphoreType.DMA,
        ],
        compiler_params=pltpu.CompilerParams(needs_layout_passes=False),
    )(_t5_bias_fill)


def _t5_bias_fill(wt_hbm, idx_hbm, out_hbm, w_v, table_v, s0, s1, s2, s3, sem):
    head = lax.axis_index("s")
    half = lax.axis_index("c")
    pltpu.sync_copy(wt_hbm, w_v)
    pltpu.sync_copy(idx_hbm, table_v)
    hbase = head * _NUM_BUCKETS

    with jax.named_scope("tbl_build"):

        @plsc.parallel_loop(0, _SHIFTS * _TBL // 16)
        def _build(t):
            base = t * 16
            iv = plsc.bitcast(table_v[pl.ds(base, 16)], jnp.int32)
            table_v[pl.ds(base, 16)] = plsc.load_gather(w_v, [iv + hbase])

    i0 = half * _HALF
    row0 = head * _LQ + i0
    ring = (s0, s1, s2, s3)

    def _fill_stage(stg, w0):
        @plsc.parallel_loop(0, _LK // 16)
        def _cp(c):
            c16 = c * 16
            src = pl.multiple_of(w0 + c16, 8)
            vals = [table_v[pl.ds(r * _TBL + src, 16)] for r in range(8)]
            for r in range(8):
                stg[r, pl.ds(c16, 16)] = vals[r]

    def _stripe(stg, s_loc):
        _fill_stage(stg, (_LQ - 8) - i0 - s_loc * 8)
        pltpu.async_copy(stg, out_hbm.at[pl.ds(row0 + s_loc * 8, 8), :], sem)

    with jax.named_scope("row_fill"):
        for b in range(4):
            _stripe(ring[b], b)

        @pl.loop(1, _HALF // 8 // 4)
        def _fill(g):
            for b in range(4):
                pltpu.make_async_copy(out_hbm.at[pl.ds(0, 8), :], ring[b], sem).wait()
                _stripe(ring[b], g * 4 + b)

        for b in range(4):
            pltpu.make_async_copy(out_hbm.at[pl.ds(0, 8), :], ring[b], sem).wait()


def kernel(lq, lk, W):
    del lq, lk
    wt = W.astype(jnp.float32).T.reshape(-1)
    idx = lax.bitcast_convert_type(jnp.asarray(_IDX_CONST), jnp.float32)
    out = _build_fill_kernel()(wt, idx)
    return out.reshape(1, _NUM_HEADS, _LQ, _LK)


# speedup vs baseline: 5.0002x; 1.0012x over previous
import functools
import math

import jax
import jax.numpy as jnp
import numpy as np
from jax import lax
from jax.experimental import pallas as pl
from jax.experimental.pallas import tpu as pltpu
from jax.experimental.pallas import tpu_sc as plsc

_NUM_BUCKETS = 32
_NUM_HEADS = 16
_LQ = 2048
_LK = 2048
_MAX_DIST = 128
_SHIFTS = 8
_TBL = 4096
_HALF = _LQ // 2


def _bucket_table() -> np.ndarray:
    d = np.arange(-(_LQ - 1), _LK, dtype=np.int32)
    rel_buckets = (d > 0).astype(np.int32) * (_NUM_BUCKETS // 2)
    rp = np.abs(d)
    max_exact = _NUM_BUCKETS // 4
    safe_rp = np.maximum(rp.astype(np.float32), np.float32(1e-9))
    large = max_exact + (
        np.log(safe_rp / max_exact)
        / math.log(_MAX_DIST / max_exact)
        * (_NUM_BUCKETS // 2 - max_exact)
    ).astype(np.int32)
    large = np.minimum(large, _NUM_BUCKETS // 2 - 1)
    return rel_buckets + np.where(rp < max_exact, rp, large)


def _shifted_index_table() -> np.ndarray:
    bucket = _bucket_table()
    u = np.arange(_TBL)
    out = np.empty((_SHIFTS, _TBL), dtype=np.int32)
    for k in range(_SHIFTS):
        dd = np.minimum(u + k, _LQ + _LK - 2)
        out[_SHIFTS - 1 - k] = bucket[dd]
    return out.reshape(-1)


_IDX_CONST = _shifted_index_table()


@functools.lru_cache(maxsize=1)
def _build_fill_kernel():
    mesh = plsc.VectorSubcoreMesh(core_axis_name="c", subcore_axis_name="s")
    return functools.partial(
        pl.kernel,
        out_type=jax.ShapeDtypeStruct((_NUM_HEADS * _LQ, _LK), jnp.float32),
        mesh=mesh,
        scratch_types=[
            pltpu.VMEM((_NUM_HEADS * _NUM_BUCKETS,), jnp.float32),
            pltpu.VMEM((_SHIFTS * _TBL,), jnp.float32),
            pltpu.VMEM((8, _LK), jnp.float32),
            pltpu.VMEM((8, _LK), jnp.float32),
            pltpu.VMEM((8, _LK), jnp.float32),
            pltpu.VMEM((8, _LK), jnp.float32),
            pltpu.SemaphoreType.DMA,
        ],
        compiler_params=pltpu.CompilerParams(needs_layout_passes=False),
    )(_t5_bias_fill)


def _t5_bias_fill(wt_hbm, idx_hbm, out_hbm, w_v, table_v, s0, s1, s2, s3, sem):
    head = lax.axis_index("s")
    half = lax.axis_index("c")
    pltpu.sync_copy(wt_hbm, w_v)
    pltpu.sync_copy(idx_hbm, table_v)
    hbase = head * _NUM_BUCKETS

    with jax.named_scope("tbl_build"):

        @plsc.parallel_loop(0, _SHIFTS * _TBL // 16)
        def _build(t):
            base = t * 16
            iv = plsc.bitcast(table_v[pl.ds(base, 16)], jnp.int32)
            table_v[pl.ds(base, 16)] = plsc.load_gather(w_v, [iv + hbase])

    i0 = half * _HALF
    row0 = head * _LQ + i0
    ring = (s0, s1, s2, s3)

    def _fill_stage(stg, w0):
        @plsc.parallel_loop(0, _LK // 16)
        def _cp(c):
            c16 = c * 16
            src = pl.multiple_of(w0 + c16, 8)
            vals = [table_v[pl.ds(r * _TBL + src, 16)] for r in range(8)]
            for r in range(8):
                stg[r, pl.ds(c16, 16)] = vals[r]

    def _stripe(stg, s_loc):
        _fill_stage(stg, (_LQ - 8) - i0 - s_loc * 8)
        pltpu.async_copy(stg, out_hbm.at[pl.ds(row0 + s_loc * 8, 8), :], sem)

    with jax.named_scope("row_fill"):
        for b in range(4):
            _stripe(ring[b], b)

        @pl.loop(1, _HALF // 8 // 4)
        def _fill(g):
            for b in range(4):
                pltpu.make_async_copy(out_hbm.at[pl.ds(0, 8), :], ring[b], sem).wait()
                _stripe(ring[b], g * 4 + b)

        for b in range(4):
            pltpu.make_async_copy(out_hbm.at[pl.ds(0, 8), :], ring[b], sem).wait()


def kernel(lq, lk, W):
    del lq, lk
    wt = W.astype(jnp.float32).T.reshape(-1)
    idx = lax.bitcast_convert_type(jnp.asarray(_IDX_CONST), jnp.float32)
    out = _build_fill_kernel()(wt, idx)
    return out.reshape(1, _NUM_HEADS, _LQ, _LK)
